# baseline (speedup 1.0000x reference)
import jax
import jax.numpy as jnp
from jax import lax
from jax.experimental import pallas as pl
from jax.experimental.pallas import tpu as pltpu
from jax.experimental.pallas import tpu_sc as plsc

_N_NODES = 10000
_N_EDGES = 320000
_D = 128
_DW = _D // 2

_NC = 2
_NS = 16
_NW = _NC * _NS
_L = 16

_EPW = _N_EDGES // _NW
_CH = 80
_NCHUNK = _EPW // _CH
_NGRP = _CH // _L


def _sc_kernel(tab_hbm, pidx_hbm, out_hbm,
               tab_sh, pch0, sidx0, didx0, pch1, sidx1, didx1,
               sr0, dr0, sr1, dr1, outc0, outc1, sem0, sem1):
    cid = lax.axis_index("c")
    sid = lax.axis_index("s")
    wid = sid * _NC + cid
    base = wid * _EPW

    @pl.when(sid == 0)
    def _():
        pltpu.sync_copy(tab_hbm, tab_sh)

    plsc.subcore_barrier()

    def start(c, pch, si, di, sr, dr, sem):
        pltpu.sync_copy(pidx_hbm.at[pl.ds(base + c * _CH, _CH)], pch)
        for i in range(_CH // _L):
            p = pch[pl.ds(i * _L, _L)]
            si[pl.ds(i * _L, _L)] = p & 0xFFFF
            di[pl.ds(i * _L, _L)] = p >> 16
        pltpu.async_copy(tab_sh.at[si], sr, sem)
        pltpu.async_copy(tab_sh.at[di], dr, sem)

    def wait(si, di, sr, dr, sem):
        pltpu.make_async_copy(tab_sh.at[si], sr, sem).wait()
        pltpu.make_async_copy(tab_sh.at[di], dr, sem).wait()

    lane_iota = lax.broadcasted_iota(jnp.int32, (_L,), 0)

    def compute(c, sr, dr, outc):
        def grp_body(g, _):
            e0 = g * _L
            chains = [jnp.zeros((_L,), jnp.float32) for _ in range(4)]
            for e in range(_L):
                row = e0 + e
                prods = []
                for j in range(4):
                    sw = plsc.bitcast(sr[row, pl.ds(j * _L, _L)], jnp.bfloat16)
                    dw = plsc.bitcast(dr[row, pl.ds(_DW + j * _L, _L)],
                                      jnp.bfloat16)
                    sa, sb = plsc.unpack(sw, format=plsc.PackFormat.INTERLEAVED,
                                         preferred_element_type=jnp.float32)
                    da, db = plsc.unpack(dw, format=plsc.PackFormat.INTERLEAVED,
                                         preferred_element_type=jnp.float32)
                    prods.append(sa * da)
                    prods.append(sb * db)
                s4 = [prods[k] + prods[k + 4] for k in range(4)]
                p = (s4[0] + s4[2]) + (s4[1] + s4[3])
                tot = jnp.sum(p)
                chains[e % 4] = jnp.where(lane_iota == e, tot, chains[e % 4])
            vec = (chains[0] + chains[1]) + (chains[2] + chains[3])
            outc[pl.ds(e0, _L)] = vec
            return 0

        lax.fori_loop(0, _NGRP, grp_body, 0)

    def step(c, pch, si, di, sr, dr, outc, sem):
        wait(si, di, sr, dr, sem)
        compute(c, sr, dr, outc)

        @pl.when(c + 2 < _NCHUNK)
        def _():
            start(c + 2, pch, si, di, sr, dr, sem)

        pltpu.sync_copy(outc, out_hbm.at[pl.ds(base + c * _CH, _CH)])

    start(0, pch0, sidx0, didx0, sr0, dr0, sem0)
    start(1, pch1, sidx1, didx1, sr1, dr1, sem1)

    def chunk_body(c, _):
        @pl.when(c % 2 == 0)
        def _():
            step(c, pch0, sidx0, didx0, sr0, dr0, outc0, sem0)

        @pl.when(c % 2 == 1)
        def _():
            step(c, pch1, sidx1, didx1, sr1, dr1, outc1, sem1)

        return 0

    lax.fori_loop(0, _NCHUNK, chunk_body, 0)


@jax.jit
def _run(tab, packed_idx):
    mesh = plsc.VectorSubcoreMesh(core_axis_name="c", subcore_axis_name="s")
    return pl.kernel(
        _sc_kernel,
        out_type=jax.ShapeDtypeStruct((_N_EDGES,), jnp.float32),
        mesh=mesh,
        compiler_params=pltpu.CompilerParams(needs_layout_passes=False),
        scratch_types=[
            pltpu.VMEM_SHARED((_N_NODES, _D), jnp.int32),
            pltpu.VMEM((_CH,), jnp.int32),
            pltpu.VMEM((_CH,), jnp.int32),
            pltpu.VMEM((_CH,), jnp.int32),
            pltpu.VMEM((_CH,), jnp.int32),
            pltpu.VMEM((_CH,), jnp.int32),
            pltpu.VMEM((_CH,), jnp.int32),
            pltpu.VMEM((_CH, _D), jnp.int32),
            pltpu.VMEM((_CH, _D), jnp.int32),
            pltpu.VMEM((_CH, _D), jnp.int32),
            pltpu.VMEM((_CH, _D), jnp.int32),
            pltpu.VMEM((_CH,), jnp.float32),
            pltpu.VMEM((_CH,), jnp.float32),
            pltpu.SemaphoreType.DMA,
            pltpu.SemaphoreType.DMA,
        ],
    )(tab, packed_idx)


def kernel(embedding_1, embedding_2, edge_label_index):
    e1 = lax.bitcast_convert_type(
        embedding_1.astype(jnp.bfloat16).reshape(_N_NODES, _DW, 2), jnp.int32)
    e2 = lax.bitcast_convert_type(
        embedding_2.astype(jnp.bfloat16).reshape(_N_NODES, _DW, 2), jnp.int32)
    tab = jnp.concatenate([e1, e2], axis=1)
    src = edge_label_index[0].astype(jnp.int32)
    dst = edge_label_index[1].astype(jnp.int32)
    packed = src | (dst << 16)
    return _run(tab, packed)


# speedup vs baseline: 1.1751x; 1.1751x over previous
import jax
import jax.numpy as jnp
from jax import lax
from jax.experimental import pallas as pl
from jax.experimental.pallas import tpu as pltpu
from jax.experimental.pallas import tpu_sc as plsc

_N_NODES = 10000
_N_EDGES = 320000
_D = 128
_DW = _D // 2

_NC = 2
_NS = 16
_NW = _NC * _NS
_L = 16

_EPW = _N_EDGES // _NW
_CH = 80
_NCHUNK = _EPW // _CH
_NGRP = _CH // _L


def _sc_kernel(tab_hbm, pidx_hbm, out_hbm,
               tab_sh, pch0, sidx0, didx0, pch1, sidx1, didx1,
               sr0, dr0, sr1, dr1, outc0, outc1,
               sem0, sem1, semi0, semi1, semo0, semo1):
    cid = lax.axis_index("c")
    sid = lax.axis_index("s")
    wid = sid * _NC + cid
    base = wid * _EPW

    @pl.when(sid == 0)
    def _():
        pltpu.sync_copy(tab_hbm, tab_sh)

    plsc.subcore_barrier()

    def unpack_idx(pch, si, di):
        for i in range(_CH // _L):
            p = pch[pl.ds(i * _L, _L)]
            si[pl.ds(i * _L, _L)] = p & 0xFFFF
            di[pl.ds(i * _L, _L)] = p >> 16

    def fire_gathers(si, di, sr, dr, sem):
        pltpu.async_copy(tab_sh.at[si], sr, sem)
        pltpu.async_copy(tab_sh.at[di], dr, sem)

    def wait_gathers(si, di, sr, dr, sem):
        pltpu.make_async_copy(tab_sh.at[si], sr, sem).wait()
        pltpu.make_async_copy(tab_sh.at[di], dr, sem).wait()

    def fire_idx(c, pch, semi):
        pltpu.async_copy(pidx_hbm.at[pl.ds(base + c * _CH, _CH)], pch, semi)

    def wait_idx(pch, semi):
        pltpu.make_async_copy(pidx_hbm.at[pl.ds(0, _CH)], pch, semi).wait()

    lane_iota = lax.broadcasted_iota(jnp.int32, (_L,), 0)

    def compute(sr, dr, outc):
        def grp_body(g, _):
            e0 = g * _L
            chains = [jnp.zeros((_L,), jnp.float32) for _ in range(4)]
            for e in range(_L):
                row = e0 + e
                prods = []
                for j in range(4):
                    sw = plsc.bitcast(sr[row, pl.ds(j * _L, _L)], jnp.bfloat16)
                    dw = plsc.bitcast(dr[row, pl.ds(_DW + j * _L, _L)],
                                      jnp.bfloat16)
                    sa, sb = plsc.unpack(sw, format=plsc.PackFormat.INTERLEAVED,
                                         preferred_element_type=jnp.float32)
                    da, db = plsc.unpack(dw, format=plsc.PackFormat.INTERLEAVED,
                                         preferred_element_type=jnp.float32)
                    prods.append(sa * da)
                    prods.append(sb * db)
                s4 = [prods[k] + prods[k + 4] for k in range(4)]
                p = (s4[0] + s4[2]) + (s4[1] + s4[3])
                tot = jnp.sum(p)
                chains[e % 4] = jnp.where(lane_iota == e, tot, chains[e % 4])
            vec = (chains[0] + chains[1]) + (chains[2] + chains[3])
            outc[pl.ds(e0, _L)] = vec
            return 0

        lax.fori_loop(0, _NGRP, grp_body, 0)

    def step(c, pch, si, di, sr, dr, outc, sem, semi, semo):
        @pl.when(c >= 2)
        def _():
            pltpu.make_async_copy(outc, out_hbm.at[pl.ds(0, _CH)], semo).wait()

        wait_gathers(si, di, sr, dr, sem)
        compute(sr, dr, outc)

        @pl.when(c + 2 < _NCHUNK)
        def _():
            wait_idx(pch, semi)
            unpack_idx(pch, si, di)
            fire_gathers(si, di, sr, dr, sem)

            @pl.when(c + 4 < _NCHUNK)
            def _():
                fire_idx(c + 4, pch, semi)

        pltpu.async_copy(outc, out_hbm.at[pl.ds(base + c * _CH, _CH)], semo)

    pltpu.sync_copy(pidx_hbm.at[pl.ds(base, _CH)], pch0)
    unpack_idx(pch0, sidx0, didx0)
    fire_gathers(sidx0, didx0, sr0, dr0, sem0)
    pltpu.sync_copy(pidx_hbm.at[pl.ds(base + _CH, _CH)], pch1)
    unpack_idx(pch1, sidx1, didx1)
    fire_gathers(sidx1, didx1, sr1, dr1, sem1)
    fire_idx(2, pch0, semi0)
    fire_idx(3, pch1, semi1)

    def chunk_body(c, _):
        @pl.when(c % 2 == 0)
        def _():
            step(c, pch0, sidx0, didx0, sr0, dr0, outc0, sem0, semi0, semo0)

        @pl.when(c % 2 == 1)
        def _():
            step(c, pch1, sidx1, didx1, sr1, dr1, outc1, sem1, semi1, semo1)

        return 0

    lax.fori_loop(0, _NCHUNK, chunk_body, 0)
    pltpu.make_async_copy(outc0, out_hbm.at[pl.ds(0, _CH)], semo0).wait()
    pltpu.make_async_copy(outc1, out_hbm.at[pl.ds(0, _CH)], semo1).wait()


@jax.jit
def _run(tab, packed_idx):
    mesh = plsc.VectorSubcoreMesh(core_axis_name="c", subcore_axis_name="s")
    return pl.kernel(
        _sc_kernel,
        out_type=jax.ShapeDtypeStruct((_N_EDGES,), jnp.float32),
        mesh=mesh,
        compiler_params=pltpu.CompilerParams(needs_layout_passes=False),
        scratch_types=[
            pltpu.VMEM_SHARED((_N_NODES, _D), jnp.int32),
            pltpu.VMEM((_CH,), jnp.int32),
            pltpu.VMEM((_CH,), jnp.int32),
            pltpu.VMEM((_CH,), jnp.int32),
            pltpu.VMEM((_CH,), jnp.int32),
            pltpu.VMEM((_CH,), jnp.int32),
            pltpu.VMEM((_CH,), jnp.int32),
            pltpu.VMEM((_CH, _D), jnp.int32),
            pltpu.VMEM((_CH, _D), jnp.int32),
            pltpu.VMEM((_CH, _D), jnp.int32),
            pltpu.VMEM((_CH, _D), jnp.int32),
            pltpu.VMEM((_CH,), jnp.float32),
            pltpu.VMEM((_CH,), jnp.float32),
            pltpu.SemaphoreType.DMA,
            pltpu.SemaphoreType.DMA,
            pltpu.SemaphoreType.DMA,
            pltpu.SemaphoreType.DMA,
            pltpu.SemaphoreType.DMA,
            pltpu.SemaphoreType.DMA,
        ],
    )(tab, packed_idx)


def kernel(embedding_1, embedding_2, edge_label_index):
    e1 = lax.bitcast_convert_type(
        embedding_1.astype(jnp.bfloat16).reshape(_N_NODES, _DW, 2), jnp.int32)
    e2 = lax.bitcast_convert_type(
        embedding_2.astype(jnp.bfloat16).reshape(_N_NODES, _DW, 2), jnp.int32)
    tab = jnp.concatenate([e1, e2], axis=1)
    src = edge_label_index[0].astype(jnp.int32)
    dst = edge_label_index[1].astype(jnp.int32)
    packed = src | (dst << 16)
    return _run(tab, packed)
